# Initial kernel scaffold; baseline (speedup 1.0000x reference)
import jax
import jax.numpy as jnp
from jax.experimental import pallas as pl


def kernel(x, edge_index, x_batch, Wl1, bl1, Wr1, Wl2, bl2, Wr2):
    raise NotImplementedError("write your pallas kernel here")


# speedup vs baseline: 6.0552x; 6.0552x over previous
import functools

import jax
import jax.numpy as jnp
from jax import lax
from jax.experimental import pallas as pl
from jax.experimental.pallas import tpu as pltpu
from jax.experimental.pallas import tpu_sc as plsc

N = 10000
D = 128
E = 320000

NPAD = 10112
NTILES = 32
CHUNK = 128
CPT = 79
NROWS = NTILES * CPT
EPAD = NROWS * CHUNK
SLAB = NPAD // 16
RCNT0 = 10016
CROWS = 79
PAD_DST0 = 10096


def _agg(zeros, feats, srcm, dstm, with_counts):
    mesh = plsc.VectorSubcoreMesh(core_axis_name="c", subcore_axis_name="s",
                                  num_cores=2, num_subcores=16)
    scratch = [
        pltpu.VMEM_SHARED((NPAD, D), jnp.float32),
        pltpu.VMEM((CHUNK,), jnp.int32),
        pltpu.VMEM((CHUNK,), jnp.int32),
        pltpu.VMEM((CHUNK, D), jnp.float32),
        pltpu.SemaphoreType.DMA,
    ]
    if with_counts:
        scratch += [
            pltpu.VMEM((CHUNK, D), jnp.float32),
            pltpu.VMEM((CHUNK,), jnp.int32),
        ]

    @functools.partial(
        pl.kernel,
        out_type=jax.ShapeDtypeStruct((2, NPAD, D), jnp.float32),
        mesh=mesh,
        scratch_types=scratch,
        compiler_params=pltpu.CompilerParams(needs_layout_passes=False),
    )
    def k(zeros_hbm, x_hbm, src_hbm, dst_hbm, out_hbm,
          accum, idx_s, idx_d, rows, sem, *cscratch):
        c = lax.axis_index("c")
        s = lax.axis_index("s")
        wid = s * 2 + c
        base = s * SLAB
        iota = lax.iota(jnp.int32, 16)
        zero16 = jnp.zeros((16,), jnp.float32)
        ones16 = jnp.ones((16,), jnp.float32)
        pltpu.sync_copy(zeros_hbm.at[pl.ds(base, SLAB)],
                        accum.at[pl.ds(base, SLAB)])
        if with_counts:
            onehot, cidx = cscratch

            def zh(i, carry):
                for j in range(D // 16):
                    onehot[i, pl.ds(j * 16, 16)] = zero16
                return carry

            lax.fori_loop(0, CHUNK, zh, 0)
        plsc.subcore_barrier()

        def step(i, carry):
            r = wid * CPT + i
            pltpu.sync_copy(src_hbm.at[r], idx_s)
            pltpu.sync_copy(dst_hbm.at[r], idx_d)
            pltpu.async_copy(x_hbm.at[idx_s], rows, sem).wait()
            pltpu.sync_copy(rows, accum.at[idx_d], add=True)
            if with_counts:
                onehot, cidx = cscratch
                for kk in range(CHUNK // 16):
                    dv = idx_d[pl.ds(kk * 16, 16)]
                    erow = kk * 16 + iota
                    colv = lax.bitwise_and(dv, D - 1)
                    plsc.store_scatter(onehot, [erow, colv], ones16)
                    cidx[pl.ds(kk * 16, 16)] = (
                        RCNT0 + lax.shift_right_logical(dv, 7))
                pltpu.sync_copy(onehot, accum.at[cidx], add=True)
                for kk in range(CHUNK // 16):
                    dv = idx_d[pl.ds(kk * 16, 16)]
                    erow = kk * 16 + iota
                    colv = lax.bitwise_and(dv, D - 1)
                    plsc.store_scatter(onehot, [erow, colv], zero16)
            return carry

        lax.fori_loop(0, CPT, step, 0)
        plsc.subcore_barrier()
        pltpu.sync_copy(accum.at[pl.ds(base, SLAB)],
                        out_hbm.at[c, pl.ds(base, SLAB)])

    return k(zeros, feats, srcm, dstm)


def _dense(p0, p1, c0, c1, xin, WlT, bl, WrT):
    BR = NPAD // 4

    def body(p0_ref, p1_ref, c0_ref, c1_ref, x_ref, wl_ref, bl_ref, wr_ref,
             o_ref):
        cnt = jnp.maximum(c0_ref[...] + c1_ref[...], 1.0)
        mean = (p0_ref[...] + p1_ref[...]) / cnt
        o_ref[...] = (
            jnp.dot(mean, wl_ref[...], preferred_element_type=jnp.float32)
            + jnp.dot(x_ref[...], wr_ref[...],
                      preferred_element_type=jnp.float32)
            + bl_ref[...])

    return pl.pallas_call(
        body,
        grid=(NPAD // BR,),
        in_specs=[
            pl.BlockSpec((BR, D), lambda i: (i, 0)),
            pl.BlockSpec((BR, D), lambda i: (i, 0)),
            pl.BlockSpec((BR, 1), lambda i: (i, 0)),
            pl.BlockSpec((BR, 1), lambda i: (i, 0)),
            pl.BlockSpec((BR, D), lambda i: (i, 0)),
            pl.BlockSpec((D, D), lambda i: (0, 0)),
            pl.BlockSpec((1, D), lambda i: (0, 0)),
            pl.BlockSpec((D, D), lambda i: (0, 0)),
        ],
        out_specs=pl.BlockSpec((BR, D), lambda i: (i, 0)),
        out_shape=jax.ShapeDtypeStruct((NPAD, D), jnp.float32),
    )(p0, p1, c0, c1, xin, WlT, bl, WrT)


def _cnt_view(p):
    return p[RCNT0:RCNT0 + CROWS].reshape(CROWS * D)[:NPAD].reshape(NPAD, 1)


def kernel(x, edge_index, x_batch, Wl1, bl1, Wr1, Wl2, bl2, Wr2):
    src = edge_index[0]
    dst = edge_index[1]
    pad_n = EPAD - E
    pad_src = jnp.arange(pad_n, dtype=jnp.int32) % N
    pad_dst = PAD_DST0 + (jnp.arange(pad_n, dtype=jnp.int32) % 16)
    srcm = jnp.concatenate([src, pad_src]).reshape(NROWS, CHUNK)
    dstm = jnp.concatenate([dst, pad_dst]).reshape(NROWS, CHUNK)
    xp = jnp.zeros((NPAD, D), jnp.float32).at[:N].set(x)
    zeros = jnp.zeros((NPAD, D), jnp.float32)

    p = _agg(zeros, xp, srcm, dstm, with_counts=True)
    c0, c1 = _cnt_view(p[0]), _cnt_view(p[1])
    h = _dense(p[0], p[1], c0, c1, xp, Wl1.T, bl1.reshape(1, D), Wr1.T)
    q = _agg(zeros, h, srcm, dstm, with_counts=False)
    out = _dense(q[0], q[1], c0, c1, h, Wl2.T, bl2.reshape(1, D), Wr2.T)
    return out[:N]
